# baseline (speedup 1.0000x reference)
import functools

import jax
import jax.numpy as jnp
from jax import lax
from jax.experimental import pallas as pl
from jax.experimental.pallas import tpu as pltpu
from jax.experimental.pallas import tpu_sc as plsc

N = 10000
D = 128
H = 8
HD = H * D

N_PAD = 10240
BLK = 1024
NB = N_PAD // BLK
E_PAD = 172032
NC, NS = 2, 16
E_TILE_A = E_PAD // NS
CH_A = E_TILE_A // 128
E_W = E_PAD // (NC * NS)
KB = 16
CH_B = E_W // KB
ROWS_T = N_PAD // NS


def _lrelu01(t):
    return jnp.where(t >= 0, t, t * 0.01)


def _relu(t):
    return jnp.maximum(t, 0.0)


def _rowmask(i, blk=BLK):
    rows = i * blk + lax.broadcasted_iota(jnp.int32, (blk, 1), 0)
    return (rows < N).astype(jnp.float32)


def _prep_body(x_ref, w_ref, as_ref, ad_ref, h_ref, al_ref, mx_ref):
    i = pl.program_id(0)
    h = jnp.dot(x_ref[...], w_ref[...], preferred_element_type=jnp.float32)
    h_ref[...] = h
    ts = (h * as_ref[...]).reshape(BLK, H, D)
    td = (h * ad_ref[...]).reshape(BLK, H, D)
    als = jnp.sum(ts, axis=2)
    ald = jnp.sum(td, axis=2)
    al_ref[...] = jnp.concatenate([als, ald], axis=1)
    blk = jnp.concatenate([jnp.max(als, axis=0, keepdims=True),
                           jnp.max(ald, axis=0, keepdims=True)], axis=0)

    @pl.when(i == 0)
    def _():
        mx_ref[...] = blk

    @pl.when(i > 0)
    def _():
        mx_ref[...] = jnp.maximum(mx_ref[...], blk)


def _gat_prep(x, W, asrc_flat, adst_flat):
    return pl.pallas_call(
        _prep_body,
        grid=(NB,),
        in_specs=[
            pl.BlockSpec((BLK, D), lambda i: (i, 0)),
            pl.BlockSpec((D, HD), lambda i: (0, 0)),
            pl.BlockSpec((1, HD), lambda i: (0, 0)),
            pl.BlockSpec((1, HD), lambda i: (0, 0)),
        ],
        out_specs=[
            pl.BlockSpec((BLK, HD), lambda i: (i, 0)),
            pl.BlockSpec((BLK, 2 * H), lambda i: (i, 0)),
            pl.BlockSpec((2, H), lambda i: (0, 0)),
        ],
        out_shape=[
            jax.ShapeDtypeStruct((N_PAD, HD), jnp.float32),
            jax.ShapeDtypeStruct((N_PAD, 2 * H), jnp.float32),
            jax.ShapeDtypeStruct((2, H), jnp.float32),
        ],
    )(x, W, asrc_flat, adst_flat)


def _o_specs():
    return [
        pl.BlockSpec((BLK, D // 2), lambda i: (i, 0)),
        pl.BlockSpec((BLK, D // 2), lambda i: (i + 2 * NB, 0)),
        pl.BlockSpec((BLK, D // 2), lambda i: (i + NB, 0)),
        pl.BlockSpec((BLK, D // 2), lambda i: (i + 3 * NB, 0)),
    ]


def _o_merge(o00, o10, o01, o11):
    return jnp.concatenate([o00[...] + o10[...], o01[...] + o11[...]],
                           axis=1) * (1.0 / H)


def _merge_body(o00, o10, o01, o11, b_ref, out_ref):
    i = pl.program_id(0)
    g = _o_merge(o00, o10, o01, o11) + b_ref[...]
    out_ref[...] = g * _rowmask(i)


def _gat_merge(o, b_row):
    return pl.pallas_call(
        _merge_body,
        grid=(NB,),
        in_specs=_o_specs() + [
            pl.BlockSpec((1, D), lambda i: (0, 0)),
        ],
        out_specs=pl.BlockSpec((BLK, D), lambda i: (i, 0)),
        out_shape=jax.ShapeDtypeStruct((N_PAD, D), jnp.float32),
    )(o, o, o, o, b_row)


def _post_body(act, o00, o10, o01, o11, x_ref, gb_ref, w1_ref, b1_ref, w2_ref,
               b2_ref, t_ref, sums_ref):
    i = pl.program_id(0)
    g = _o_merge(o00, o10, o01, o11) + gb_ref[...]
    tin = g + x_ref[...]
    u = act(jnp.dot(tin, w1_ref[...], preferred_element_type=jnp.float32)
            + b1_ref[...])
    t = (jnp.dot(u, w2_ref[...], preferred_element_type=jnp.float32)
         + b2_ref[...] + x_ref[...])
    t = t * _rowmask(i)
    t_ref[...] = t
    blk = jnp.concatenate([jnp.sum(t, axis=0, keepdims=True),
                           jnp.sum(t * t, axis=0, keepdims=True)], axis=0)

    @pl.when(i == 0)
    def _():
        sums_ref[...] = blk

    @pl.when(i > 0)
    def _():
        sums_ref[...] = sums_ref[...] + blk


def _gat_post(act, o, x, gb_row, w1, b1_row, w2, b2_row):
    return pl.pallas_call(
        functools.partial(_post_body, act),
        grid=(NB,),
        in_specs=_o_specs() + [
            pl.BlockSpec((BLK, D), lambda i: (i, 0)),
            pl.BlockSpec((1, D), lambda i: (0, 0)),
            pl.BlockSpec((D, 5 * D), lambda i: (0, 0)),
            pl.BlockSpec((1, 5 * D), lambda i: (0, 0)),
            pl.BlockSpec((5 * D, D), lambda i: (0, 0)),
            pl.BlockSpec((1, D), lambda i: (0, 0)),
        ],
        out_specs=[
            pl.BlockSpec((BLK, D), lambda i: (i, 0)),
            pl.BlockSpec((2, D), lambda i: (0, 0)),
        ],
        out_shape=[
            jax.ShapeDtypeStruct((N_PAD, D), jnp.float32),
            jax.ShapeDtypeStruct((2, D), jnp.float32),
        ],
    )(o, o, o, o, x, gb_row, w1, b1_row, w2, b2_row)


def _bn_body(t_ref, sums_ref, g_ref, b_ref, out_ref):
    i = pl.program_id(0)
    mu = sums_ref[0:1] * (1.0 / N)
    var = sums_ref[1:2] * (1.0 / N) - mu * mu
    inv = lax.rsqrt(var + 1e-5)
    y = (t_ref[...] - mu) * inv * g_ref[...] + b_ref[...]
    out_ref[...] = y * _rowmask(i)


def _bn_apply(t, sums, g_row, b_row):
    return pl.pallas_call(
        _bn_body,
        grid=(NB,),
        in_specs=[
            pl.BlockSpec((BLK, D), lambda i: (i, 0)),
            pl.BlockSpec((2, D), lambda i: (0, 0)),
            pl.BlockSpec((1, D), lambda i: (0, 0)),
            pl.BlockSpec((1, D), lambda i: (0, 0)),
        ],
        out_specs=pl.BlockSpec((BLK, D), lambda i: (i, 0)),
        out_shape=jax.ShapeDtypeStruct((N_PAD, D), jnp.float32),
    )(t, sums, g_row, b_row)


def _embed_body(x_ref, w1_ref, b1_ref, w2_ref, b2_ref, out_ref):
    i = pl.program_id(0)
    u = _lrelu01(jnp.dot(x_ref[...], w1_ref[...],
                         preferred_element_type=jnp.float32) + b1_ref[...])
    y = jnp.dot(u, w2_ref[...], preferred_element_type=jnp.float32) + b2_ref[...]
    out_ref[...] = y * _rowmask(i)


def _embed(x, w1, b1_row, w2, b2_row):
    return pl.pallas_call(
        _embed_body,
        grid=(NB,),
        in_specs=[
            pl.BlockSpec((BLK, D), lambda i: (i, 0)),
            pl.BlockSpec((D, 5 * D), lambda i: (0, 0)),
            pl.BlockSpec((1, 5 * D), lambda i: (0, 0)),
            pl.BlockSpec((5 * D, D), lambda i: (0, 0)),
            pl.BlockSpec((1, D), lambda i: (0, 0)),
        ],
        out_specs=pl.BlockSpec((BLK, D), lambda i: (i, 0)),
        out_shape=jax.ShapeDtypeStruct((N_PAD, D), jnp.float32),
    )(x, w1, b1_row, w2, b2_row)


def _dec_prep_body(x_ref, w8_ref, tab_ref, mx_ref):
    i = pl.program_id(0)
    tab = jnp.dot(x_ref[...], w8_ref[...], preferred_element_type=jnp.float32)
    tab_ref[...] = tab
    blk = jnp.max(tab, axis=0, keepdims=True)

    @pl.when(i == 0)
    def _():
        mx_ref[...] = blk

    @pl.when(i > 0)
    def _():
        mx_ref[...] = jnp.maximum(mx_ref[...], blk)


def _dec_prep(x, w16):
    return pl.pallas_call(
        _dec_prep_body,
        grid=(NB,),
        in_specs=[
            pl.BlockSpec((BLK, D), lambda i: (i, 0)),
            pl.BlockSpec((D, 16), lambda i: (0, 0)),
        ],
        out_specs=[
            pl.BlockSpec((BLK, 16), lambda i: (i, 0)),
            pl.BlockSpec((1, 16), lambda i: (0, 0)),
        ],
        out_shape=[
            jax.ShapeDtypeStruct((N_PAD, 16), jnp.float32),
            jax.ShapeDtypeStruct((1, 16), jnp.float32),
        ],
    )(x, w16)


def _dec_final_body(p0_ref, p1_ref, b_ref, out_ref):
    acc = p0_ref[...] + p1_ref[...]
    den = acc[:, 0:1]
    num = acc[:, 1:2]
    out_ref[...] = num / (den + 1e-16) + b_ref[...]


def _dec_final(part, b11):
    return pl.pallas_call(
        _dec_final_body,
        grid=(NB,),
        in_specs=[
            pl.BlockSpec((BLK, 16), lambda i: (i, 0)),
            pl.BlockSpec((BLK, 16), lambda i: (i + NB, 0)),
            pl.BlockSpec((1, 1), lambda i: (0, 0)),
        ],
        out_specs=pl.BlockSpec((BLK, 1), lambda i: (i, 0)),
        out_shape=jax.ShapeDtypeStruct((N_PAD, 1), jnp.float32),
    )(part, part, b11)


@functools.cache
def _mesh():
    return plsc.VectorSubcoreMesh(core_axis_name="c", subcore_axis_name="s",
                                  num_cores=NC, num_subcores=NS)


DH = D // 2


@functools.cache
def _gat_sc_kernel():
    return pl.kernel(
        _gat_sc_body,
        out_type=jax.ShapeDtypeStruct((2 * NC * N_PAD, DH), jnp.float32),
        mesh=_mesh(),
        compiler_params=pltpu.CompilerParams(use_tc_tiling_on_sc=False),
        scratch_types=(
        [pltpu.VMEM((512,), jnp.int32)]
        + 2 * [pltpu.VMEM((128,), jnp.int32)]
        + 2 * [pltpu.VMEM((128,), jnp.int32)]
        + 2 * [pltpu.VMEM((128, 16), jnp.float32)]
        + 2 * [pltpu.VMEM((128, 16), jnp.float32)]
        + 2 * [pltpu.VMEM((128, 16), jnp.float32)]
        + [pltpu.VMEM((4 * KB,), jnp.int32)]
        + 2 * [pltpu.VMEM((KB,), jnp.int32)]
        + 2 * [pltpu.VMEM((H * KB,), jnp.int32)]
        + 2 * [pltpu.VMEM((H * KB, DH), jnp.float32)]
        + 2 * [pltpu.VMEM((2 * KB, 16), jnp.float32)]
        + 2 * [pltpu.VMEM((KB, 16), jnp.float32)]
        + [
            pltpu.VMEM((KB, 16), jnp.float32),
            pltpu.VMEM((KB, DH), jnp.float32),
            pltpu.VMEM((16,), jnp.float32),
            pltpu.VMEM_SHARED((N_PAD, 16), jnp.float32),
            pltpu.VMEM_SHARED((N_PAD, DH), jnp.float32),
            pltpu.SemaphoreType.DMA,
            pltpu.SemaphoreType.DMA,
        ]),
    )


def _perm(v, idx16):
    return jnp.take_along_axis(v, idx16, axis=0, mode='promise_in_bounds')


def _gat_sc_body(h_hbm, al_hbm, esdA_hbm, esdB_hbm, sh_hbm, o_hbm,
            iA2, iAs0, iAs1, iAd0, iAd1, gsA0, gsA1, gdA0, gdA1, pA0, pA1,
            iBsd2, iBd0, iBd1, idx80, idx81, ghh0, ghh1, gsgd0, gsgd1,
            dnB0, dnB1, alB, msg, shv, den, outacc, semA, semB):
    bufA = ((iAs0, iAd0, gsA0, gdA0, pA0), (iAs1, iAd1, gsA1, gdA1, pA1))
    bufB = ((iBd0, idx80, ghh0, gsgd0, dnB0), (iBd1, idx81, ghh1, gsgd1, dnB1))
    pA = pA0
    c = lax.axis_index("c")
    s = lax.axis_index("s")
    iota = lax.iota(jnp.int32, 16)
    rot8 = jnp.bitwise_and(iota + 8, 15)
    z16 = jnp.zeros((16,), jnp.float32)

    pltpu.sync_copy(sh_hbm, shv)
    shvec = shv[...]

    for k in range(KB):
        for j in range(DH // 16):
            msg[k, pl.ds(j * 16, 16)] = z16

    def zpA(k, carry):
        pA[k] = z16
        return carry

    lax.fori_loop(0, 128, zpA, 0)

    rb = s * ROWS_T
    for q in range(ROWS_T // 128):
        pltpu.sync_copy(pA, den.at[pl.ds(rb + q * 128, 128)])
    for q in range(ROWS_T // KB):
        pltpu.sync_copy(msg, outacc.at[pl.ds(rb + q * KB, KB)])
    plsc.subcore_barrier()

    def bodyA(i, carry):
        chA = s * CH_A + 2 * i
        pltpu.sync_copy(esdA_hbm.at[pl.ds(chA * 256, 512)], iA2)
        for b, (iAs, iAd, gsA, gdA, pAb) in enumerate(bufA):
            for q in range(8):
                iAs[pl.ds(q * 16, 16)] = iA2[pl.ds(b * 256 + q * 16, 16)]
                iAd[pl.ds(q * 16, 16)] = iA2[pl.ds(b * 256 + 128 + q * 16, 16)]
            pltpu.async_copy(al_hbm.at[iAs], gsA, semA)
            pltpu.async_copy(al_hbm.at[iAd], gdA, semA)
        for b, (iAs, iAd, gsA, gdA, pAb) in enumerate(bufA):
            pltpu.make_async_copy(al_hbm.at[iAs], gsA, semA).wait()
            pltpu.make_async_copy(al_hbm.at[iAd], gdA, semA).wait()

            def inner(j, carry2, gsA=gsA, gdA=gdA, pAb=pAb):
                e16 = gsA[j] + _perm(gdA[j], rot8)
                e16 = jnp.where(e16 >= 0, e16, e16 * 0.2)
                pAb[j] = jnp.exp(e16 - shvec)
                return carry2

            lax.fori_loop(0, 128, inner, 0)
            pltpu.sync_copy(pAb, den.at[iAd], add=True)
        return carry

    lax.fori_loop(0, CH_A // 2, bodyA, 0)
    plsc.subcore_barrier()

    wbase = c * (E_PAD // 2) + s * E_W

    def make_bodyB(r):
        def bodyB(i, carry):
            ch = wbase // KB + 2 * i
            pltpu.sync_copy(esdB_hbm.at[pl.ds(ch * 2 * KB, 4 * KB)], iBsd2)
            for b, (iBd, idx8, ghh, gsgd, dnB) in enumerate(bufB):
                o2 = b * 2 * KB
                for q in range(KB // 16):
                    v = iBsd2[pl.ds(o2 + q * 16, 16)] * (2 * H)
                    d = iBsd2[pl.ds(o2 + KB + q * 16, 16)]
                    iBd[pl.ds(q * 16, 16)] = d
                    for h in range(H):
                        idx8[pl.ds(h * KB + q * 16, 16)] = v + (2 * h + r)
                pltpu.async_copy(
                    al_hbm.at[iBsd2.at[pl.ds(o2, 2 * KB)]], gsgd, semA)
                for g in range(H * KB // 128):
                    pltpu.async_copy(h_hbm.at[idx8.at[pl.ds(g * 128, 128)]],
                                     ghh.at[pl.ds(g * 128, 128)], semA)
                pltpu.async_copy(den.at[iBd], dnB, semB)
            for b, (iBd, idx8, ghh, gsgd, dnB) in enumerate(bufB):
                o2 = b * 2 * KB
                pltpu.make_async_copy(
                    al_hbm.at[iBsd2.at[pl.ds(o2, 2 * KB)]], gsgd, semA).wait()
                for g in range(H * KB // 128):
                    pltpu.make_async_copy(
                        h_hbm.at[idx8.at[pl.ds(g * 128, 128)]],
                        ghh.at[pl.ds(g * 128, 128)], semA).wait()
                pltpu.make_async_copy(den.at[iBd], dnB, semB).wait()

                for j in range(KB):
                    e16 = gsgd[j] + _perm(gsgd[KB + j], rot8)
                    e16 = jnp.where(e16 >= 0, e16, e16 * 0.2)
                    p16 = jnp.exp(e16 - shvec)
                    alB[j] = p16 / (dnB[j] + 1e-16)

                for k in range(KB):
                    av = alB[k]
                    accs = [None] * (DH // 16)
                    for h in range(H):
                        ab = _perm(av, jnp.full((16,), h, jnp.int32))
                        for j in range(DH // 16):
                            hv = ghh[h * KB + k, pl.ds(j * 16, 16)]
                            if h == 0:
                                accs[j] = ab * hv
                            else:
                                accs[j] = accs[j] + ab * hv
                    for j in range(DH // 16):
                        msg[k, pl.ds(j * 16, 16)] = accs[j]

                pltpu.sync_copy(msg, outacc.at[iBd], add=True)
            return carry
        return bodyB

    for r in range(2):
        lax.fori_loop(0, CH_B // 2, make_bodyB(r), 0)
        plsc.subcore_barrier()
        obase = (2 * c + r) * N_PAD
        for q in range(ROWS_T // KB):
            pltpu.sync_copy(outacc.at[pl.ds(rb + q * KB, KB)], msg)
            pltpu.sync_copy(msg, o_hbm.at[pl.ds(obase + rb + q * KB, KB)])
        if r == 0:
            for k in range(KB):
                for j in range(DH // 16):
                    msg[k, pl.ds(j * 16, 16)] = z16
            for q in range(ROWS_T // KB):
                pltpu.sync_copy(msg, outacc.at[pl.ds(rb + q * KB, KB)])
            plsc.subcore_barrier()


@functools.cache
def _dec_sc_kernel():
    return pl.kernel(
        _dec_sc_body,
        out_type=jax.ShapeDtypeStruct((NC * N_PAD, 16), jnp.float32),
        mesh=_mesh(),
        compiler_params=pltpu.CompilerParams(use_tc_tiling_on_sc=False),
        scratch_types=[
            pltpu.VMEM((128,), jnp.int32),
            pltpu.VMEM((128,), jnp.int32),
            pltpu.VMEM((128, 16), jnp.float32),
            pltpu.VMEM((128, 16), jnp.float32),
            pltpu.VMEM((128, 16), jnp.float32),
            pltpu.VMEM((16,), jnp.float32),
            pltpu.VMEM_SHARED((N_PAD, 16), jnp.float32),
            pltpu.SemaphoreType.DMA,
        ],
    )


def _dec_sc_body(tab_hbm, src_hbm, dst_hbm, sh_hbm, part_hbm,
                 iDs, iDd, gsD, gdD, pbuf, shv, acc, sem):
    c = lax.axis_index("c")
    s = lax.axis_index("s")
    iota = lax.iota(jnp.int32, 16)
    rot1 = jnp.bitwise_and(iota + 1, 15)
    z16 = jnp.zeros((16,), jnp.float32)
    pltpu.sync_copy(sh_hbm, shv)
    shvec = shv[...]

    def zp(k, carry):
        pbuf[k] = z16
        return carry

    lax.fori_loop(0, 128, zp, 0)
    rb = s * ROWS_T
    for q in range(ROWS_T // 128):
        pltpu.sync_copy(pbuf, acc.at[pl.ds(rb + q * 128, 128)])
    plsc.subcore_barrier()

    wbase = c * (E_PAD // 2) + s * E_W

    def body(cb, carry):
        eb = wbase + cb * 128
        pltpu.sync_copy(src_hbm.at[pl.ds(eb, 128)], iDs)
        pltpu.sync_copy(dst_hbm.at[pl.ds(eb, 128)], iDd)
        pltpu.async_copy(tab_hbm.at[iDs], gsD, sem).wait()
        pltpu.async_copy(tab_hbm.at[iDd], gdD, sem).wait()

        def inner(j, carry2):
            gs = gsD[j]
            gd1 = _perm(gdD[j], rot1)
            e16 = jnp.where(iota == 0, gs + gd1, z16)
            e16 = jnp.where(e16 >= 0, e16, e16 * 0.2)
            p16 = jnp.exp(e16 - shvec)
            ps = _perm(p16, jnp.zeros((16,), jnp.int32))
            hs = _perm(gs, jnp.full((16,), 2, jnp.int32))
            pbuf[j] = jnp.where(iota == 0, ps,
                                jnp.where(iota == 1, ps * hs, z16))
            return carry2

        lax.fori_loop(0, 128, inner, 0)
        pltpu.sync_copy(pbuf, acc.at[iDd], add=True)
        return carry

    lax.fori_loop(0, E_W // 128, body, 0)
    plsc.subcore_barrier()
    for q in range(ROWS_T // 128):
        pltpu.sync_copy(acc.at[pl.ds(rb + q * 128, 128)], pbuf)
        pltpu.sync_copy(pbuf, part_hbm.at[pl.ds(c * N_PAD + rb + q * 128, 128)])


def _row(v):
    return v.reshape(1, -1)


def _gat_layer(x, W, asrc, adst, esdA, esdB):
    h, al, mx = _gat_prep(x, W, _row(asrc.reshape(-1)), _row(adst.reshape(-1)))
    sh = mx[0] + mx[1]
    sh = jnp.where(sh >= 0, sh, sh * 0.2)
    sh16 = jnp.concatenate([sh, sh])
    return _gat_sc_kernel()(h.reshape(N_PAD * H * 2, DH), al, esdA, esdB, sh16)


def kernel(x, edge_index, params):
    p = params
    loops = jnp.arange(N, dtype=edge_index.dtype)
    pad = jnp.full((E_PAD - 160000 - N,), N, dtype=edge_index.dtype)
    srcE = jnp.concatenate([edge_index[0], loops, pad])
    dstE = jnp.concatenate([edge_index[1], loops, pad])
    esdA = jnp.concatenate([srcE.reshape(-1, 128), dstE.reshape(-1, 128)],
                           axis=1).reshape(-1)
    esdB = jnp.concatenate([srcE.reshape(-1, KB), dstE.reshape(-1, KB)],
                           axis=1).reshape(-1)

    xp = jnp.zeros((N_PAD, D), jnp.float32).at[:N].set(x)
    x_out = _embed(xp, p['emb_w1'], _row(p['emb_b1']),
                   p['emb_w2'], _row(p['emb_b2']))

    acts = {1: _lrelu01, 2: _relu, 3: _relu}
    for _ in range(3):
        o = _gat_layer(x_out, p['gat1_W'], p['gat1_asrc'], p['gat1_adst'],
                       esdA, esdB)
        x_out = _gat_merge(o, _row(p['gat1_b']))
        cur = x_out
        for i in (1, 2, 3):
            o = _gat_layer(cur, p[f'gat{i}_W'], p[f'gat{i}_asrc'],
                           p[f'gat{i}_adst'], esdA, esdB)
            t, sums = _gat_post(acts[i], o, cur, _row(p[f'gat{i}_b']),
                                p[f'ff{i}_w1'], _row(p[f'ff{i}_b1']),
                                p[f'ff{i}_w2'], _row(p[f'ff{i}_b2']))
            cur = _bn_apply(t, sums, _row(p[f'bn{i}_g']), _row(p[f'bn{i}_b']))
        x_out = cur

    w16 = jnp.zeros((D, 16), jnp.float32)
    w16 = w16.at[:, 0].set(p['dec_W'][:, 0] * p['dec_asrc'][0, 0])
    w16 = w16.at[:, 1].set(p['dec_W'][:, 0] * p['dec_adst'][0, 0])
    w16 = w16.at[:, 2].set(p['dec_W'][:, 0])
    tab, mx = _dec_prep(x_out, w16)
    shd = mx[0, 0] + mx[0, 1]
    shd = jnp.where(shd >= 0, shd, shd * 0.2)
    sh16 = jnp.full((16,), 1.0, jnp.float32) * shd
    part = _dec_sc_kernel()(tab, srcE, dstE, sh16)
    out = _dec_final(part, p['dec_b'].reshape(1, 1))
    return out[:N]


# speedup vs baseline: 1.0923x; 1.0923x over previous
import functools

import jax
import jax.numpy as jnp
from jax import lax
from jax.experimental import pallas as pl
from jax.experimental.pallas import tpu as pltpu
from jax.experimental.pallas import tpu_sc as plsc

N = 10000
D = 128
H = 8
HD = H * D

N_PAD = 10240
BLK = 1024
NB = N_PAD // BLK
E_PAD = 172032
NC, NS = 2, 16
E_TILE_A = E_PAD // NS
CH_A = E_TILE_A // 128
E_W = E_PAD // (NC * NS)
KB = 32
CH_B = E_W // KB
ROWS_T = N_PAD // NS


def _lrelu01(t):
    return jnp.where(t >= 0, t, t * 0.01)


def _relu(t):
    return jnp.maximum(t, 0.0)


def _rowmask(i, blk=BLK):
    rows = i * blk + lax.broadcasted_iota(jnp.int32, (blk, 1), 0)
    return (rows < N).astype(jnp.float32)


def _prep_body(x_ref, w_ref, as_ref, ad_ref, h_ref, al_ref, mx_ref):
    i = pl.program_id(0)
    h = jnp.dot(x_ref[...], w_ref[...], preferred_element_type=jnp.float32)
    h_ref[...] = h
    ts = (h * as_ref[...]).reshape(BLK, H, D)
    td = (h * ad_ref[...]).reshape(BLK, H, D)
    als = jnp.sum(ts, axis=2)
    ald = jnp.sum(td, axis=2)
    al_ref[...] = jnp.concatenate([als, ald], axis=1)
    blk = jnp.concatenate([jnp.max(als, axis=0, keepdims=True),
                           jnp.max(ald, axis=0, keepdims=True)], axis=0)

    @pl.when(i == 0)
    def _():
        mx_ref[...] = blk

    @pl.when(i > 0)
    def _():
        mx_ref[...] = jnp.maximum(mx_ref[...], blk)


def _gat_prep(x, W, asrc_flat, adst_flat):
    return pl.pallas_call(
        _prep_body,
        grid=(NB,),
        in_specs=[
            pl.BlockSpec((BLK, D), lambda i: (i, 0)),
            pl.BlockSpec((D, HD), lambda i: (0, 0)),
            pl.BlockSpec((1, HD), lambda i: (0, 0)),
            pl.BlockSpec((1, HD), lambda i: (0, 0)),
        ],
        out_specs=[
            pl.BlockSpec((BLK, HD), lambda i: (i, 0)),
            pl.BlockSpec((BLK, 2 * H), lambda i: (i, 0)),
            pl.BlockSpec((2, H), lambda i: (0, 0)),
        ],
        out_shape=[
            jax.ShapeDtypeStruct((N_PAD, HD), jnp.float32),
            jax.ShapeDtypeStruct((N_PAD, 2 * H), jnp.float32),
            jax.ShapeDtypeStruct((2, H), jnp.float32),
        ],
    )(x, W, asrc_flat, adst_flat)


def _o_specs():
    return [
        pl.BlockSpec((BLK, D // 2), lambda i: (i, 0)),
        pl.BlockSpec((BLK, D // 2), lambda i: (i + 2 * NB, 0)),
        pl.BlockSpec((BLK, D // 2), lambda i: (i + NB, 0)),
        pl.BlockSpec((BLK, D // 2), lambda i: (i + 3 * NB, 0)),
    ]


def _o_merge(o00, o10, o01, o11):
    return jnp.concatenate([o00[...] + o10[...], o01[...] + o11[...]],
                           axis=1) * (1.0 / H)


def _merge_body(o00, o10, o01, o11, b_ref, out_ref):
    i = pl.program_id(0)
    g = _o_merge(o00, o10, o01, o11) + b_ref[...]
    out_ref[...] = g * _rowmask(i)


def _gat_merge(o, b_row):
    return pl.pallas_call(
        _merge_body,
        grid=(NB,),
        in_specs=_o_specs() + [
            pl.BlockSpec((1, D), lambda i: (0, 0)),
        ],
        out_specs=pl.BlockSpec((BLK, D), lambda i: (i, 0)),
        out_shape=jax.ShapeDtypeStruct((N_PAD, D), jnp.float32),
    )(o, o, o, o, b_row)


def _post_body(act, o00, o10, o01, o11, x_ref, gb_ref, w1_ref, b1_ref, w2_ref,
               b2_ref, t_ref, sums_ref):
    i = pl.program_id(0)
    g = _o_merge(o00, o10, o01, o11) + gb_ref[...]
    tin = g + x_ref[...]
    u = act(jnp.dot(tin, w1_ref[...], preferred_element_type=jnp.float32)
            + b1_ref[...])
    t = (jnp.dot(u, w2_ref[...], preferred_element_type=jnp.float32)
         + b2_ref[...] + x_ref[...])
    t = t * _rowmask(i)
    t_ref[...] = t
    blk = jnp.concatenate([jnp.sum(t, axis=0, keepdims=True),
                           jnp.sum(t * t, axis=0, keepdims=True)], axis=0)

    @pl.when(i == 0)
    def _():
        sums_ref[...] = blk

    @pl.when(i > 0)
    def _():
        sums_ref[...] = sums_ref[...] + blk


def _gat_post(act, o, x, gb_row, w1, b1_row, w2, b2_row):
    return pl.pallas_call(
        functools.partial(_post_body, act),
        grid=(NB,),
        in_specs=_o_specs() + [
            pl.BlockSpec((BLK, D), lambda i: (i, 0)),
            pl.BlockSpec((1, D), lambda i: (0, 0)),
            pl.BlockSpec((D, 5 * D), lambda i: (0, 0)),
            pl.BlockSpec((1, 5 * D), lambda i: (0, 0)),
            pl.BlockSpec((5 * D, D), lambda i: (0, 0)),
            pl.BlockSpec((1, D), lambda i: (0, 0)),
        ],
        out_specs=[
            pl.BlockSpec((BLK, D), lambda i: (i, 0)),
            pl.BlockSpec((2, D), lambda i: (0, 0)),
        ],
        out_shape=[
            jax.ShapeDtypeStruct((N_PAD, D), jnp.float32),
            jax.ShapeDtypeStruct((2, D), jnp.float32),
        ],
    )(o, o, o, o, x, gb_row, w1, b1_row, w2, b2_row)


def _bn_body(t_ref, sums_ref, g_ref, b_ref, out_ref):
    i = pl.program_id(0)
    mu = sums_ref[0:1] * (1.0 / N)
    var = sums_ref[1:2] * (1.0 / N) - mu * mu
    inv = lax.rsqrt(var + 1e-5)
    y = (t_ref[...] - mu) * inv * g_ref[...] + b_ref[...]
    out_ref[...] = y * _rowmask(i)


def _bn_apply(t, sums, g_row, b_row):
    return pl.pallas_call(
        _bn_body,
        grid=(NB,),
        in_specs=[
            pl.BlockSpec((BLK, D), lambda i: (i, 0)),
            pl.BlockSpec((2, D), lambda i: (0, 0)),
            pl.BlockSpec((1, D), lambda i: (0, 0)),
            pl.BlockSpec((1, D), lambda i: (0, 0)),
        ],
        out_specs=pl.BlockSpec((BLK, D), lambda i: (i, 0)),
        out_shape=jax.ShapeDtypeStruct((N_PAD, D), jnp.float32),
    )(t, sums, g_row, b_row)


def _embed_body(x_ref, w1_ref, b1_ref, w2_ref, b2_ref, out_ref):
    i = pl.program_id(0)
    u = _lrelu01(jnp.dot(x_ref[...], w1_ref[...],
                         preferred_element_type=jnp.float32) + b1_ref[...])
    y = jnp.dot(u, w2_ref[...], preferred_element_type=jnp.float32) + b2_ref[...]
    out_ref[...] = y * _rowmask(i)


def _embed(x, w1, b1_row, w2, b2_row):
    return pl.pallas_call(
        _embed_body,
        grid=(NB,),
        in_specs=[
            pl.BlockSpec((BLK, D), lambda i: (i, 0)),
            pl.BlockSpec((D, 5 * D), lambda i: (0, 0)),
            pl.BlockSpec((1, 5 * D), lambda i: (0, 0)),
            pl.BlockSpec((5 * D, D), lambda i: (0, 0)),
            pl.BlockSpec((1, D), lambda i: (0, 0)),
        ],
        out_specs=pl.BlockSpec((BLK, D), lambda i: (i, 0)),
        out_shape=jax.ShapeDtypeStruct((N_PAD, D), jnp.float32),
    )(x, w1, b1_row, w2, b2_row)


def _dec_prep_body(x_ref, w8_ref, tab_ref, mx_ref):
    i = pl.program_id(0)
    tab = jnp.dot(x_ref[...], w8_ref[...], preferred_element_type=jnp.float32)
    tab_ref[...] = tab
    blk = jnp.max(tab, axis=0, keepdims=True)

    @pl.when(i == 0)
    def _():
        mx_ref[...] = blk

    @pl.when(i > 0)
    def _():
        mx_ref[...] = jnp.maximum(mx_ref[...], blk)


def _dec_prep(x, w16):
    return pl.pallas_call(
        _dec_prep_body,
        grid=(NB,),
        in_specs=[
            pl.BlockSpec((BLK, D), lambda i: (i, 0)),
            pl.BlockSpec((D, 16), lambda i: (0, 0)),
        ],
        out_specs=[
            pl.BlockSpec((BLK, 16), lambda i: (i, 0)),
            pl.BlockSpec((1, 16), lambda i: (0, 0)),
        ],
        out_shape=[
            jax.ShapeDtypeStruct((N_PAD, 16), jnp.float32),
            jax.ShapeDtypeStruct((1, 16), jnp.float32),
        ],
    )(x, w16)


def _dec_final_body(p0_ref, p1_ref, b_ref, out_ref):
    acc = p0_ref[...] + p1_ref[...]
    den = acc[:, 0:1]
    num = acc[:, 1:2]
    out_ref[...] = num / (den + 1e-16) + b_ref[...]


def _dec_final(part, b11):
    return pl.pallas_call(
        _dec_final_body,
        grid=(NB,),
        in_specs=[
            pl.BlockSpec((BLK, 16), lambda i: (i, 0)),
            pl.BlockSpec((BLK, 16), lambda i: (i + NB, 0)),
            pl.BlockSpec((1, 1), lambda i: (0, 0)),
        ],
        out_specs=pl.BlockSpec((BLK, 1), lambda i: (i, 0)),
        out_shape=jax.ShapeDtypeStruct((N_PAD, 1), jnp.float32),
    )(part, part, b11)


@functools.cache
def _mesh():
    return plsc.VectorSubcoreMesh(core_axis_name="c", subcore_axis_name="s",
                                  num_cores=NC, num_subcores=NS)


DH = D // 2


@functools.cache
def _gat_sc_kernel():
    return pl.kernel(
        _gat_sc_body,
        out_type=jax.ShapeDtypeStruct((2 * NC * N_PAD, DH), jnp.float32),
        mesh=_mesh(),
        compiler_params=pltpu.CompilerParams(use_tc_tiling_on_sc=False),
        scratch_types=(
        [pltpu.VMEM((512,), jnp.int32)]
        + 2 * [pltpu.VMEM((128,), jnp.int32)]
        + 2 * [pltpu.VMEM((128,), jnp.int32)]
        + 2 * [pltpu.VMEM((128, 16), jnp.float32)]
        + 2 * [pltpu.VMEM((128, 16), jnp.float32)]
        + 2 * [pltpu.VMEM((128, 16), jnp.float32)]
        + [pltpu.VMEM((4 * KB,), jnp.int32)]
        + 2 * [pltpu.VMEM((KB,), jnp.int32)]
        + 2 * [pltpu.VMEM((H * KB,), jnp.int32)]
        + 2 * [pltpu.VMEM((H * KB, DH), jnp.float32)]
        + 2 * [pltpu.VMEM((2 * KB, 16), jnp.float32)]
        + 2 * [pltpu.VMEM((KB, 16), jnp.float32)]
        + [
            pltpu.VMEM((KB, 16), jnp.float32),
            pltpu.VMEM((KB, DH), jnp.float32),
            pltpu.VMEM((16,), jnp.float32),
            pltpu.VMEM_SHARED((N_PAD, 16), jnp.float32),
            pltpu.VMEM_SHARED((N_PAD, DH), jnp.float32),
            pltpu.SemaphoreType.DMA,
            pltpu.SemaphoreType.DMA,
        ]),
    )


def _perm(v, idx16):
    return jnp.take_along_axis(v, idx16, axis=0, mode='promise_in_bounds')


def _gat_sc_body(h_hbm, al_hbm, esdA_hbm, esdB_hbm, sh_hbm, o_hbm,
            iA2, iAs0, iAs1, iAd0, iAd1, gsA0, gsA1, gdA0, gdA1, pA0, pA1,
            iBsd2, iBd0, iBd1, idx80, idx81, ghh0, ghh1, gsgd0, gsgd1,
            dnB0, dnB1, alB, msg, shv, den, outacc, semA, semB):
    bufA = ((iAs0, iAd0, gsA0, gdA0, pA0), (iAs1, iAd1, gsA1, gdA1, pA1))
    bufB = ((iBd0, idx80, ghh0, gsgd0, dnB0), (iBd1, idx81, ghh1, gsgd1, dnB1))
    pA = pA0
    c = lax.axis_index("c")
    s = lax.axis_index("s")
    iota = lax.iota(jnp.int32, 16)
    rot8 = jnp.bitwise_and(iota + 8, 15)
    z16 = jnp.zeros((16,), jnp.float32)

    pltpu.sync_copy(sh_hbm, shv)
    shvec = shv[...]

    for k in range(KB):
        for j in range(DH // 16):
            msg[k, pl.ds(j * 16, 16)] = z16

    def zpA(k, carry):
        pA[k] = z16
        return carry

    lax.fori_loop(0, 128, zpA, 0)

    rb = s * ROWS_T
    for q in range(ROWS_T // 128):
        pltpu.sync_copy(pA, den.at[pl.ds(rb + q * 128, 128)])
    for q in range(ROWS_T // KB):
        pltpu.sync_copy(msg, outacc.at[pl.ds(rb + q * KB, KB)])
    plsc.subcore_barrier()

    def bodyA(i, carry):
        chA = s * CH_A + i
        pltpu.sync_copy(esdA_hbm.at[pl.ds(chA * 256, 256)],
                        iA2.at[pl.ds(0, 256)])
        for b, (iAs, iAd, gsA, gdA, pAb) in enumerate(bufA[:1]):
            for q in range(8):
                iAs[pl.ds(q * 16, 16)] = iA2[pl.ds(b * 256 + q * 16, 16)]
                iAd[pl.ds(q * 16, 16)] = iA2[pl.ds(b * 256 + 128 + q * 16, 16)]
            pltpu.async_copy(al_hbm.at[iAs], gsA, semA)
            pltpu.async_copy(al_hbm.at[iAd], gdA, semA)
        for b, (iAs, iAd, gsA, gdA, pAb) in enumerate(bufA[:1]):
            pltpu.make_async_copy(al_hbm.at[iAs], gsA, semA).wait()
            pltpu.make_async_copy(al_hbm.at[iAd], gdA, semA).wait()

            def inner(j, carry2, gsA=gsA, gdA=gdA, pAb=pAb):
                e16 = gsA[j] + _perm(gdA[j], rot8)
                e16 = jnp.where(e16 >= 0, e16, e16 * 0.2)
                pAb[j] = jnp.exp(e16 - shvec)
                return carry2

            lax.fori_loop(0, 128, inner, 0)
            pltpu.sync_copy(pAb, den.at[iAd], add=True)
        return carry

    lax.fori_loop(0, CH_A, bodyA, 0)
    plsc.subcore_barrier()

    wbase = c * (E_PAD // 2) + s * E_W

    def make_bodyB(r):
        def bodyB(i, carry):
            ch = wbase // KB + 2 * i
            pltpu.sync_copy(esdB_hbm.at[pl.ds(ch * 2 * KB, 4 * KB)], iBsd2)
            for b, (iBd, idx8, ghh, gsgd, dnB) in enumerate(bufB):
                o2 = b * 2 * KB
                for q in range(KB // 16):
                    v = iBsd2[pl.ds(o2 + q * 16, 16)] * (2 * H)
                    d = iBsd2[pl.ds(o2 + KB + q * 16, 16)]
                    iBd[pl.ds(q * 16, 16)] = d
                    for h in range(H):
                        idx8[pl.ds(h * KB + q * 16, 16)] = v + (2 * h + r)
                pltpu.async_copy(
                    al_hbm.at[iBsd2.at[pl.ds(o2, 2 * KB)]], gsgd, semA)
                for g in range(H * KB // 128):
                    pltpu.async_copy(h_hbm.at[idx8.at[pl.ds(g * 128, 128)]],
                                     ghh.at[pl.ds(g * 128, 128)], semA)
                pltpu.async_copy(den.at[iBd], dnB, semB)
            for b, (iBd, idx8, ghh, gsgd, dnB) in enumerate(bufB):
                o2 = b * 2 * KB
                pltpu.make_async_copy(
                    al_hbm.at[iBsd2.at[pl.ds(o2, 2 * KB)]], gsgd, semA).wait()
                for g in range(H * KB // 128):
                    pltpu.make_async_copy(
                        h_hbm.at[idx8.at[pl.ds(g * 128, 128)]],
                        ghh.at[pl.ds(g * 128, 128)], semA).wait()
                pltpu.make_async_copy(den.at[iBd], dnB, semB).wait()

                for j in range(KB):
                    e16 = gsgd[j] + _perm(gsgd[KB + j], rot8)
                    e16 = jnp.where(e16 >= 0, e16, e16 * 0.2)
                    p16 = jnp.exp(e16 - shvec)
                    alB[j] = p16 / (dnB[j] + 1e-16)

                for k in range(KB):
                    av = alB[k]
                    accs = [None] * (DH // 16)
                    for h in range(H):
                        ab = _perm(av, jnp.full((16,), h, jnp.int32))
                        for j in range(DH // 16):
                            hv = ghh[h * KB + k, pl.ds(j * 16, 16)]
                            if h == 0:
                                accs[j] = ab * hv
                            else:
                                accs[j] = accs[j] + ab * hv
                    for j in range(DH // 16):
                        msg[k, pl.ds(j * 16, 16)] = accs[j]

                pltpu.sync_copy(msg, outacc.at[iBd], add=True)
            return carry
        return bodyB

    for r in range(2):
        lax.fori_loop(0, CH_B // 2, make_bodyB(r), 0)
        plsc.subcore_barrier()
        obase = (2 * c + r) * N_PAD
        for q in range(ROWS_T // KB):
            pltpu.sync_copy(outacc.at[pl.ds(rb + q * KB, KB)], msg)
            pltpu.sync_copy(msg, o_hbm.at[pl.ds(obase + rb + q * KB, KB)])
        if r == 0:
            for k in range(KB):
                for j in range(DH // 16):
                    msg[k, pl.ds(j * 16, 16)] = z16
            for q in range(ROWS_T // KB):
                pltpu.sync_copy(msg, outacc.at[pl.ds(rb + q * KB, KB)])
            plsc.subcore_barrier()


@functools.cache
def _dec_sc_kernel():
    return pl.kernel(
        _dec_sc_body,
        out_type=jax.ShapeDtypeStruct((NC * N_PAD, 16), jnp.float32),
        mesh=_mesh(),
        compiler_params=pltpu.CompilerParams(use_tc_tiling_on_sc=False),
        scratch_types=[
            pltpu.VMEM((128,), jnp.int32),
            pltpu.VMEM((128,), jnp.int32),
            pltpu.VMEM((128, 16), jnp.float32),
            pltpu.VMEM((128, 16), jnp.float32),
            pltpu.VMEM((128, 16), jnp.float32),
            pltpu.VMEM((16,), jnp.float32),
            pltpu.VMEM_SHARED((N_PAD, 16), jnp.float32),
            pltpu.SemaphoreType.DMA,
        ],
    )


def _dec_sc_body(tab_hbm, src_hbm, dst_hbm, sh_hbm, part_hbm,
                 iDs, iDd, gsD, gdD, pbuf, shv, acc, sem):
    c = lax.axis_index("c")
    s = lax.axis_index("s")
    iota = lax.iota(jnp.int32, 16)
    rot1 = jnp.bitwise_and(iota + 1, 15)
    z16 = jnp.zeros((16,), jnp.float32)
    pltpu.sync_copy(sh_hbm, shv)
    shvec = shv[...]

    def zp(k, carry):
        pbuf[k] = z16
        return carry

    lax.fori_loop(0, 128, zp, 0)
    rb = s * ROWS_T
    for q in range(ROWS_T // 128):
        pltpu.sync_copy(pbuf, acc.at[pl.ds(rb + q * 128, 128)])
    plsc.subcore_barrier()

    wbase = c * (E_PAD // 2) + s * E_W

    def body(cb, carry):
        eb = wbase + cb * 128
        pltpu.sync_copy(src_hbm.at[pl.ds(eb, 128)], iDs)
        pltpu.sync_copy(dst_hbm.at[pl.ds(eb, 128)], iDd)
        pltpu.async_copy(tab_hbm.at[iDs], gsD, sem).wait()
        pltpu.async_copy(tab_hbm.at[iDd], gdD, sem).wait()

        def inner(j, carry2):
            gs = gsD[j]
            gd1 = _perm(gdD[j], rot1)
            e16 = jnp.where(iota == 0, gs + gd1, z16)
            e16 = jnp.where(e16 >= 0, e16, e16 * 0.2)
            p16 = jnp.exp(e16 - shvec)
            ps = _perm(p16, jnp.zeros((16,), jnp.int32))
            hs = _perm(gs, jnp.full((16,), 2, jnp.int32))
            pbuf[j] = jnp.where(iota == 0, ps,
                                jnp.where(iota == 1, ps * hs, z16))
            return carry2

        lax.fori_loop(0, 128, inner, 0)
        pltpu.sync_copy(pbuf, acc.at[iDd], add=True)
        return carry

    lax.fori_loop(0, E_W // 128, body, 0)
    plsc.subcore_barrier()
    for q in range(ROWS_T // 128):
        pltpu.sync_copy(acc.at[pl.ds(rb + q * 128, 128)], pbuf)
        pltpu.sync_copy(pbuf, part_hbm.at[pl.ds(c * N_PAD + rb + q * 128, 128)])


def _row(v):
    return v.reshape(1, -1)


def _gat_layer(x, W, asrc, adst, esdA, esdB):
    h, al, mx = _gat_prep(x, W, _row(asrc.reshape(-1)), _row(adst.reshape(-1)))
    sh = mx[0] + mx[1]
    sh = jnp.where(sh >= 0, sh, sh * 0.2)
    sh16 = jnp.concatenate([sh, sh])
    return _gat_sc_kernel()(h.reshape(N_PAD * H * 2, DH), al, esdA, esdB, sh16)


def kernel(x, edge_index, params):
    p = params
    loops = jnp.arange(N, dtype=edge_index.dtype)
    pad = jnp.full((E_PAD - 160000 - N,), N, dtype=edge_index.dtype)
    srcE = jnp.concatenate([edge_index[0], loops, pad])
    dstE = jnp.concatenate([edge_index[1], loops, pad])
    esdA = jnp.concatenate([srcE.reshape(-1, 128), dstE.reshape(-1, 128)],
                           axis=1).reshape(-1)
    esdB = jnp.concatenate([srcE.reshape(-1, KB), dstE.reshape(-1, KB)],
                           axis=1).reshape(-1)

    xp = jnp.zeros((N_PAD, D), jnp.float32).at[:N].set(x)
    x_out = _embed(xp, p['emb_w1'], _row(p['emb_b1']),
                   p['emb_w2'], _row(p['emb_b2']))

    acts = {1: _lrelu01, 2: _relu, 3: _relu}
    for _ in range(3):
        o = _gat_layer(x_out, p['gat1_W'], p['gat1_asrc'], p['gat1_adst'],
                       esdA, esdB)
        x_out = _gat_merge(o, _row(p['gat1_b']))
        cur = x_out
        for i in (1, 2, 3):
            o = _gat_layer(cur, p[f'gat{i}_W'], p[f'gat{i}_asrc'],
                           p[f'gat{i}_adst'], esdA, esdB)
            t, sums = _gat_post(acts[i], o, cur, _row(p[f'gat{i}_b']),
                                p[f'ff{i}_w1'], _row(p[f'ff{i}_b1']),
                                p[f'ff{i}_w2'], _row(p[f'ff{i}_b2']))
            cur = _bn_apply(t, sums, _row(p[f'bn{i}_g']), _row(p[f'bn{i}_b']))
        x_out = cur

    w16 = jnp.zeros((D, 16), jnp.float32)
    w16 = w16.at[:, 0].set(p['dec_W'][:, 0] * p['dec_asrc'][0, 0])
    w16 = w16.at[:, 1].set(p['dec_W'][:, 0] * p['dec_adst'][0, 0])
    w16 = w16.at[:, 2].set(p['dec_W'][:, 0])
    tab, mx = _dec_prep(x_out, w16)
    shd = mx[0, 0] + mx[0, 1]
    shd = jnp.where(shd >= 0, shd, shd * 0.2)
    sh16 = jnp.full((16,), 1.0, jnp.float32) * shd
    part = _dec_sc_kernel()(tab, srcE, dstE, sh16)
    out = _dec_final(part, p['dec_b'].reshape(1, 1))
    return out[:N]
